# Initial kernel scaffold; baseline (speedup 1.0000x reference)
import jax
import jax.numpy as jnp
from jax import lax
from jax.experimental import pallas as pl
from jax.experimental.pallas import tpu as pltpu
from jax.experimental.pallas import tpu_sc as plsc

N = 10000
E = 320000
D = 128

NC = 2
NS = 16
NW = NC * NS
CHUNK = 128
CPW = 80
EPW = CHUNK * CPW
E_PAD = EPW * NW
ROWS_PER_TILE = N // NS
ZCOPY = 125

_VREGS_PER_ROW = D // 16


def _sc_agg_kernel(h_hbm, row_hbm, col_hbm, ew_hbm, out_hbm,
                   col_v, row_v, ew_v, rows_v, acc_sh):
  c = lax.axis_index("c")
  s = lax.axis_index("s")
  wid = s * NC + c

  def zbody(i, carry):
    for k in range(_VREGS_PER_ROW):
      rows_v[i, pl.ds(k * 16, 16)] = jnp.zeros((16,), jnp.float32)
    return carry
  lax.fori_loop(0, ZCOPY, zbody, 0)

  for t in range(ROWS_PER_TILE // ZCOPY):
    pltpu.sync_copy(rows_v.at[pl.ds(0, ZCOPY)],
                    acc_sh.at[pl.ds(s * ROWS_PER_TILE + t * ZCOPY, ZCOPY)])
  plsc.subcore_barrier()

  pltpu.sync_copy(col_hbm.at[pl.ds(wid * CPW, CPW)], col_v)
  pltpu.sync_copy(row_hbm.at[pl.ds(wid * CPW, CPW)], row_v)
  pltpu.sync_copy(ew_hbm.at[pl.ds(wid * CPW, CPW)], ew_v)

  def chunk_body(j, carry):
    pltpu.sync_copy(h_hbm.at[col_v.at[j]], rows_v)

    def ebody(e, c2):
      w = ew_v[j, e]
      for k in range(_VREGS_PER_ROW):
        sl = pl.ds(k * 16, 16)
        rows_v[e, sl] = rows_v[e, sl] * w
      return c2
    lax.fori_loop(0, CHUNK, ebody, 0)

    pltpu.sync_copy(rows_v, acc_sh.at[row_v.at[j]], add=True)
    return carry
  lax.fori_loop(0, CPW, chunk_body, 0)

  plsc.subcore_barrier()

  for t in range(ROWS_PER_TILE // ZCOPY):
    r0 = s * ROWS_PER_TILE + t * ZCOPY
    pltpu.sync_copy(acc_sh.at[pl.ds(r0, ZCOPY)],
                    out_hbm.at[c, pl.ds(r0, ZCOPY)])


def _sc_aggregate(h, row2d, col2d, ew2d):
  mesh = plsc.VectorSubcoreMesh(core_axis_name="c", subcore_axis_name="s")
  return pl.kernel(
      _sc_agg_kernel,
      out_type=jax.ShapeDtypeStruct((NC, N, D), jnp.float32),
      mesh=mesh,
      scratch_types=[
          pltpu.VMEM((CPW, CHUNK), jnp.int32),
          pltpu.VMEM((CPW, CHUNK), jnp.int32),
          pltpu.VMEM((CPW, CHUNK), jnp.float32),
          pltpu.VMEM((CHUNK, D), jnp.float32),
          pltpu.VMEM_SHARED((N, D), jnp.float32),
      ],
  )(h, row2d, col2d, ew2d)


BR = 1000


def _tc_proj_kernel(x_ref, w_ref, b_ref, o_ref):
  o_ref[...] = (
      jax.lax.dot_general(
          x_ref[...], w_ref[...], (((1,), (0,)), ((), ())),
          precision=lax.Precision.HIGHEST,
          preferred_element_type=jnp.float32)
      + b_ref[...])


def _tc_proj(x, w, b):
  return pl.pallas_call(
      _tc_proj_kernel,
      out_shape=jax.ShapeDtypeStruct((N, D), jnp.float32),
      grid=(N // BR,),
      in_specs=[
          pl.BlockSpec((BR, D), lambda i: (i, 0)),
          pl.BlockSpec((D, D), lambda i: (0, 0)),
          pl.BlockSpec((1, D), lambda i: (0, 0)),
      ],
      out_specs=pl.BlockSpec((BR, D), lambda i: (i, 0)),
  )(x, w, b.reshape(1, D))


def _tc_layer_kernel(p0_ref, p1_ref, w_ref, b_ref, g_ref, be_ref, o_ref):
  agg = p0_ref[...] + p1_ref[...]
  y = jax.lax.dot_general(
      agg, w_ref[...], (((1,), (0,)), ((), ())),
      precision=lax.Precision.HIGHEST,
      preferred_element_type=jnp.float32) + b_ref[...]
  y = jnp.maximum(y, 0.0)
  mu = jnp.mean(y, axis=-1, keepdims=True)
  var = jnp.mean((y - mu) ** 2, axis=-1, keepdims=True)
  o_ref[...] = (y - mu) * lax.rsqrt(var + 1e-5) * g_ref[...] + be_ref[...]


def _tc_layer_final_kernel(p0_ref, p1_ref, w_ref, b_ref, g_ref, be_ref,
                           res_ref, emb_ref, o_ref):
  agg = p0_ref[...] + p1_ref[...]
  y = jax.lax.dot_general(
      agg, w_ref[...], (((1,), (0,)), ((), ())),
      precision=lax.Precision.HIGHEST,
      preferred_element_type=jnp.float32) + b_ref[...]
  y = jnp.maximum(y, 0.0)
  mu = jnp.mean(y, axis=-1, keepdims=True)
  var = jnp.mean((y - mu) ** 2, axis=-1, keepdims=True)
  ln = (y - mu) * lax.rsqrt(var + 1e-5) * g_ref[...] + be_ref[...]
  o_ref[...] = ln + res_ref[...] + emb_ref[...]


def _tc_layer(parts, w, b, gamma, beta, residual=None, emb=None):
  p0 = parts[0]
  p1 = parts[1]
  row_spec = pl.BlockSpec((BR, D), lambda i: (i, 0))
  vec_spec = pl.BlockSpec((1, D), lambda i: (0, 0))
  mat_spec = pl.BlockSpec((D, D), lambda i: (0, 0))
  if residual is None:
    return pl.pallas_call(
        _tc_layer_kernel,
        out_shape=jax.ShapeDtypeStruct((N, D), jnp.float32),
        grid=(N // BR,),
        in_specs=[row_spec, row_spec, mat_spec, vec_spec, vec_spec, vec_spec],
        out_specs=row_spec,
    )(p0, p1, w, b.reshape(1, D), gamma.reshape(1, D), beta.reshape(1, D))
  return pl.pallas_call(
      _tc_layer_final_kernel,
      out_shape=jax.ShapeDtypeStruct((N, D), jnp.float32),
      grid=(N // BR,),
      in_specs=[row_spec, row_spec, mat_spec, vec_spec, vec_spec, vec_spec,
                row_spec, row_spec],
      out_specs=row_spec,
  )(p0, p1, w, b.reshape(1, D), gamma.reshape(1, D), beta.reshape(1, D),
    residual, emb)


@jax.jit
def kernel(node_features, edge_index, edge_weight, W_proj, b_proj,
           W1, b1, W2, b2, gamma, beta, item_emb):
  row = edge_index[0]
  col = edge_index[1]
  pad = E_PAD - E
  zi = jnp.zeros((pad,), jnp.int32)
  row2d = jnp.concatenate([row, zi]).reshape(NW * CPW, CHUNK)
  col2d = jnp.concatenate([col, zi]).reshape(NW * CPW, CHUNK)
  ew2d = jnp.concatenate(
      [edge_weight, jnp.zeros((pad,), jnp.float32)]).reshape(NW * CPW, CHUNK)

  h = _tc_proj(node_features, W_proj, b_proj)
  residual = h

  parts = _sc_aggregate(h, row2d, col2d, ew2d)
  h = _tc_layer(parts, W1, b1, gamma, beta)

  parts = _sc_aggregate(h, row2d, col2d, ew2d)
  out = _tc_layer(parts, W2, b2, gamma, beta, residual=residual, emb=item_emb)
  return out


# speedup vs baseline: 2.9468x; 2.9468x over previous
import jax
import jax.numpy as jnp
from jax import lax
from jax.experimental import pallas as pl
from jax.experimental.pallas import tpu as pltpu
from jax.experimental.pallas import tpu_sc as plsc

N = 10000
E = 320000
D = 128

NC = 2
NS = 16
NW = NC * NS
CHUNK = 128
CPW = 80
EPW = CHUNK * CPW
E_PAD = EPW * NW
N_ACC = 10240
ROWS_PER_TILE = N_ACC // NS
ZCOPY = 128

_VREGS_PER_ROW = D // 16


def _sc_agg_kernel(h_hbm, row_hbm, col_hbm, ew_hbm, out_hbm,
                   col_v, row_v, ew_v, rows_v, acc_sh):
  c = lax.axis_index("c")
  s = lax.axis_index("s")
  wid = s * NC + c

  def zbody(i, carry):
    for k in range(_VREGS_PER_ROW):
      rows_v[i, pl.ds(k * 16, 16)] = jnp.zeros((16,), jnp.float32)
    return carry
  lax.fori_loop(0, CHUNK, zbody, 0)

  for t in range(ROWS_PER_TILE // ZCOPY):
    pltpu.sync_copy(rows_v,
                    acc_sh.at[pl.ds(s * ROWS_PER_TILE + t * ZCOPY, ZCOPY)])
  plsc.subcore_barrier()

  pltpu.sync_copy(col_hbm.at[pl.ds(wid * CPW, CPW)], col_v)
  pltpu.sync_copy(row_hbm.at[pl.ds(wid * CPW, CPW)], row_v)
  pltpu.sync_copy(ew_hbm.at[pl.ds(wid * CPW, CPW)], ew_v)

  def chunk_body(j, carry):
    pltpu.sync_copy(h_hbm.at[col_v.at[j]], rows_v)

    def gbody(g, c2):
      base = g * 16
      w16 = ew_v[j, pl.ds(base, 16)]
      for i in range(16):
        w = w16[i]
        for k in range(_VREGS_PER_ROW):
          sl = pl.ds(k * 16, 16)
          rows_v[base + i, sl] = rows_v[base + i, sl] * w
      return c2
    lax.fori_loop(0, CHUNK // 16, gbody, 0)

    pltpu.sync_copy(rows_v, acc_sh.at[row_v.at[j]], add=True)
    return carry
  lax.fori_loop(0, CPW, chunk_body, 0)

  plsc.subcore_barrier()

  for t in range(ROWS_PER_TILE // ZCOPY):
    r0 = s * ROWS_PER_TILE + t * ZCOPY
    pltpu.sync_copy(acc_sh.at[pl.ds(r0, ZCOPY)],
                    out_hbm.at[c, pl.ds(r0, ZCOPY)])


def _sc_aggregate(h, row2d, col2d, ew2d):
  mesh = plsc.VectorSubcoreMesh(core_axis_name="c", subcore_axis_name="s")
  return pl.kernel(
      _sc_agg_kernel,
      out_type=jax.ShapeDtypeStruct((NC, N_ACC, D), jnp.float32),
      mesh=mesh,
      scratch_types=[
          pltpu.VMEM((CPW, CHUNK), jnp.int32),
          pltpu.VMEM((CPW, CHUNK), jnp.int32),
          pltpu.VMEM((CPW, CHUNK), jnp.float32),
          pltpu.VMEM((CHUNK, D), jnp.float32),
          pltpu.VMEM_SHARED((N_ACC, D), jnp.float32),
      ],
  )(h, row2d, col2d, ew2d)


BR = 1000


def _tc_proj_kernel(x_ref, w_ref, b_ref, o_ref):
  o_ref[...] = (
      jax.lax.dot_general(
          x_ref[...], w_ref[...], (((1,), (0,)), ((), ())),
          precision=lax.Precision.HIGHEST,
          preferred_element_type=jnp.float32)
      + b_ref[...])


def _tc_proj(x, w, b):
  return pl.pallas_call(
      _tc_proj_kernel,
      out_shape=jax.ShapeDtypeStruct((N, D), jnp.float32),
      grid=(N // BR,),
      in_specs=[
          pl.BlockSpec((BR, D), lambda i: (i, 0)),
          pl.BlockSpec((D, D), lambda i: (0, 0)),
          pl.BlockSpec((1, D), lambda i: (0, 0)),
      ],
      out_specs=pl.BlockSpec((BR, D), lambda i: (i, 0)),
  )(x, w, b.reshape(1, D))


def _tc_layer_kernel(p0_ref, p1_ref, w_ref, b_ref, g_ref, be_ref, o_ref):
  agg = p0_ref[...] + p1_ref[...]
  y = jax.lax.dot_general(
      agg, w_ref[...], (((1,), (0,)), ((), ())),
      precision=lax.Precision.HIGHEST,
      preferred_element_type=jnp.float32) + b_ref[...]
  y = jnp.maximum(y, 0.0)
  mu = jnp.mean(y, axis=-1, keepdims=True)
  var = jnp.mean((y - mu) ** 2, axis=-1, keepdims=True)
  o_ref[...] = (y - mu) * lax.rsqrt(var + 1e-5) * g_ref[...] + be_ref[...]


def _tc_layer_final_kernel(p0_ref, p1_ref, w_ref, b_ref, g_ref, be_ref,
                           res_ref, emb_ref, o_ref):
  agg = p0_ref[...] + p1_ref[...]
  y = jax.lax.dot_general(
      agg, w_ref[...], (((1,), (0,)), ((), ())),
      precision=lax.Precision.HIGHEST,
      preferred_element_type=jnp.float32) + b_ref[...]
  y = jnp.maximum(y, 0.0)
  mu = jnp.mean(y, axis=-1, keepdims=True)
  var = jnp.mean((y - mu) ** 2, axis=-1, keepdims=True)
  ln = (y - mu) * lax.rsqrt(var + 1e-5) * g_ref[...] + be_ref[...]
  o_ref[...] = ln + res_ref[...] + emb_ref[...]


def _tc_layer(parts, w, b, gamma, beta, residual=None, emb=None):
  p0 = parts[0, :N]
  p1 = parts[1, :N]
  row_spec = pl.BlockSpec((BR, D), lambda i: (i, 0))
  vec_spec = pl.BlockSpec((1, D), lambda i: (0, 0))
  mat_spec = pl.BlockSpec((D, D), lambda i: (0, 0))
  if residual is None:
    return pl.pallas_call(
        _tc_layer_kernel,
        out_shape=jax.ShapeDtypeStruct((N, D), jnp.float32),
        grid=(N // BR,),
        in_specs=[row_spec, row_spec, mat_spec, vec_spec, vec_spec, vec_spec],
        out_specs=row_spec,
    )(p0, p1, w, b.reshape(1, D), gamma.reshape(1, D), beta.reshape(1, D))
  return pl.pallas_call(
      _tc_layer_final_kernel,
      out_shape=jax.ShapeDtypeStruct((N, D), jnp.float32),
      grid=(N // BR,),
      in_specs=[row_spec, row_spec, mat_spec, vec_spec, vec_spec, vec_spec,
                row_spec, row_spec],
      out_specs=row_spec,
  )(p0, p1, w, b.reshape(1, D), gamma.reshape(1, D), beta.reshape(1, D),
    residual, emb)


@jax.jit
def kernel(node_features, edge_index, edge_weight, W_proj, b_proj,
           W1, b1, W2, b2, gamma, beta, item_emb):
  row = edge_index[0]
  col = edge_index[1]
  pad = E_PAD - E
  zi = jnp.zeros((pad,), jnp.int32)
  row2d = jnp.concatenate([row, zi]).reshape(NW * CPW, CHUNK)
  col2d = jnp.concatenate([col, zi]).reshape(NW * CPW, CHUNK)
  ew2d = jnp.concatenate(
      [edge_weight, jnp.zeros((pad,), jnp.float32)]).reshape(NW * CPW, CHUNK)

  h = _tc_proj(node_features, W_proj, b_proj)
  residual = h

  parts = _sc_aggregate(h, row2d, col2d, ew2d)
  h = _tc_layer(parts, W1, b1, gamma, beta)

  parts = _sc_aggregate(h, row2d, col2d, ew2d)
  out = _tc_layer(parts, W2, b2, gamma, beta, residual=residual, emb=item_emb)
  return out


# speedup vs baseline: 3.0945x; 1.0501x over previous
import jax
import jax.numpy as jnp
from jax import lax
from jax.experimental import pallas as pl
from jax.experimental.pallas import tpu as pltpu
from jax.experimental.pallas import tpu_sc as plsc

N = 10000
E = 320000
D = 128

NC = 2
NS = 16
NW = NC * NS
CHUNK = 64
CPW = 160
STAGES = 4
HALF = CPW // STAGES
EPW = CHUNK * CPW
E_PAD = EPW * NW
N_ACC = 10240
ROWS_PER_TILE = N_ACC // NS
ZCOPY = CHUNK

_VREGS_PER_ROW = D // 16


def _sc_agg_kernel(h_hbm, row_hbm, col_hbm, ew_hbm, out_hbm,
                   col_v, row_v, w0, w1, g0, g1, s0, s1,
                   wsem0, wsem1, gsem0, gsem1, ssem0, ssem1, acc_sh):
  c = lax.axis_index("c")
  s = lax.axis_index("s")
  wid = s * NC + c

  def zbody(i, carry):
    for k in range(_VREGS_PER_ROW):
      g0[i, pl.ds(k * 16, 16)] = jnp.zeros((16,), jnp.float32)
    return carry
  lax.fori_loop(0, CHUNK, zbody, 0)

  for t in range(ROWS_PER_TILE // ZCOPY):
    pltpu.sync_copy(g0,
                    acc_sh.at[pl.ds(s * ROWS_PER_TILE + t * ZCOPY, ZCOPY)])
  plsc.subcore_barrier()

  def scale(gbuf, sbuf, wbuf):
    def ebody(e, c2):
      wv = wbuf[e // 8, pl.ds((e % 8) * 16, 16)]
      for k in range(_VREGS_PER_ROW):
        sl = pl.ds(k * 16, 16)
        sbuf[e, sl] = gbuf[e, sl] * wv
      return c2
    lax.fori_loop(0, CHUNK, ebody, 0)

  def start_wfetch(jg, wbuf, wsem):
    pltpu.make_async_copy(ew_hbm.at[jg], wbuf, wsem).start()

  def wait_wfetch(jg, wbuf, wsem):
    pltpu.make_async_copy(ew_hbm.at[jg], wbuf, wsem).wait()

  def start_gather(j, gbuf, gsem):
    pltpu.make_async_copy(h_hbm.at[col_v.at[j]], gbuf, gsem).start()

  def wait_gather(j, gbuf, gsem):
    pltpu.make_async_copy(h_hbm.at[col_v.at[j]], gbuf, gsem).wait()

  def start_scatter(j, sbuf, ssem):
    pltpu.make_async_copy(sbuf, acc_sh.at[row_v.at[j]], ssem).start(add=True)

  def wait_scatter(j, sbuf, ssem):
    pltpu.make_async_copy(sbuf, acc_sh.at[row_v.at[j]], ssem).wait()

  def stage_body(half, carry0):
    base_c = wid * CPW + half * HALF
    pltpu.sync_copy(col_hbm.at[pl.ds(base_c, HALF)], col_v)
    pltpu.sync_copy(row_hbm.at[pl.ds(base_c, HALF)], row_v)

    start_gather(0, g0, gsem0)
    start_gather(1, g1, gsem1)
    start_wfetch(base_c, w0, wsem0)
    start_wfetch(base_c + 1, w1, wsem1)

    def chunk_body(jj, carry):
      for p, (gbuf, gsem, sbuf, ssem, wbuf, wsem) in enumerate(
          ((g0, gsem0, s0, ssem0, w0, wsem0),
           (g1, gsem1, s1, ssem1, w1, wsem1))):
        j = 2 * jj + p
        wait_gather(j, gbuf, gsem)
        wait_wfetch(base_c + j, wbuf, wsem)

        @pl.when(jj >= 1)
        def _():
          wait_scatter(j - 2, sbuf, ssem)

        scale(gbuf, sbuf, wbuf)
        start_scatter(j, sbuf, ssem)

        @pl.when(jj < HALF // 2 - 1)
        def _():
          start_gather(j + 2, gbuf, gsem)
          start_wfetch(base_c + j + 2, wbuf, wsem)
      return carry
    lax.fori_loop(0, HALF // 2, chunk_body, 0)

    wait_scatter(HALF - 2, s0, ssem0)
    wait_scatter(HALF - 1, s1, ssem1)
    return carry0
  lax.fori_loop(0, STAGES, stage_body, 0)

  plsc.subcore_barrier()

  for t in range(ROWS_PER_TILE // ZCOPY):
    r0 = s * ROWS_PER_TILE + t * ZCOPY
    pltpu.sync_copy(acc_sh.at[pl.ds(r0, ZCOPY)],
                    out_hbm.at[c, pl.ds(r0, ZCOPY)])


def _sc_aggregate(h, row2d, col2d, ew_exp):
  mesh = plsc.VectorSubcoreMesh(core_axis_name="c", subcore_axis_name="s")
  return pl.kernel(
      _sc_agg_kernel,
      out_type=jax.ShapeDtypeStruct((NC, N_ACC, D), jnp.float32),
      mesh=mesh,
      scratch_types=[
          pltpu.VMEM((HALF, CHUNK), jnp.int32),
          pltpu.VMEM((HALF, CHUNK), jnp.int32),
          pltpu.VMEM((8, 128), jnp.float32),
          pltpu.VMEM((8, 128), jnp.float32),
          pltpu.VMEM((CHUNK, D), jnp.float32),
          pltpu.VMEM((CHUNK, D), jnp.float32),
          pltpu.VMEM((CHUNK, D), jnp.float32),
          pltpu.VMEM((CHUNK, D), jnp.float32),
          pltpu.SemaphoreType.DMA,
          pltpu.SemaphoreType.DMA,
          pltpu.SemaphoreType.DMA,
          pltpu.SemaphoreType.DMA,
          pltpu.SemaphoreType.DMA,
          pltpu.SemaphoreType.DMA,
          pltpu.VMEM_SHARED((N_ACC, D), jnp.float32),
      ],
  )(h, row2d, col2d, ew_exp)


BR = 1000


def _tc_proj_kernel(x_ref, w_ref, b_ref, o_ref):
  o_ref[...] = (
      jax.lax.dot_general(
          x_ref[...], w_ref[...], (((1,), (0,)), ((), ())),
          precision=lax.Precision.HIGHEST,
          preferred_element_type=jnp.float32)
      + b_ref[...])


def _tc_proj(x, w, b):
  return pl.pallas_call(
      _tc_proj_kernel,
      out_shape=jax.ShapeDtypeStruct((N, D), jnp.float32),
      grid=(N // BR,),
      in_specs=[
          pl.BlockSpec((BR, D), lambda i: (i, 0)),
          pl.BlockSpec((D, D), lambda i: (0, 0)),
          pl.BlockSpec((1, D), lambda i: (0, 0)),
      ],
      out_specs=pl.BlockSpec((BR, D), lambda i: (i, 0)),
  )(x, w, b.reshape(1, D))


def _tc_layer_kernel(p0_ref, p1_ref, w_ref, b_ref, g_ref, be_ref, o_ref):
  agg = p0_ref[...] + p1_ref[...]
  y = jax.lax.dot_general(
      agg, w_ref[...], (((1,), (0,)), ((), ())),
      precision=lax.Precision.HIGHEST,
      preferred_element_type=jnp.float32) + b_ref[...]
  y = jnp.maximum(y, 0.0)
  mu = jnp.mean(y, axis=-1, keepdims=True)
  var = jnp.mean((y - mu) ** 2, axis=-1, keepdims=True)
  o_ref[...] = (y - mu) * lax.rsqrt(var + 1e-5) * g_ref[...] + be_ref[...]


def _tc_layer_final_kernel(p0_ref, p1_ref, w_ref, b_ref, g_ref, be_ref,
                           res_ref, emb_ref, o_ref):
  agg = p0_ref[...] + p1_ref[...]
  y = jax.lax.dot_general(
      agg, w_ref[...], (((1,), (0,)), ((), ())),
      precision=lax.Precision.HIGHEST,
      preferred_element_type=jnp.float32) + b_ref[...]
  y = jnp.maximum(y, 0.0)
  mu = jnp.mean(y, axis=-1, keepdims=True)
  var = jnp.mean((y - mu) ** 2, axis=-1, keepdims=True)
  ln = (y - mu) * lax.rsqrt(var + 1e-5) * g_ref[...] + be_ref[...]
  o_ref[...] = ln + res_ref[...] + emb_ref[...]


def _tc_layer(parts, w, b, gamma, beta, residual=None, emb=None):
  p0 = parts[0, :N]
  p1 = parts[1, :N]
  row_spec = pl.BlockSpec((BR, D), lambda i: (i, 0))
  vec_spec = pl.BlockSpec((1, D), lambda i: (0, 0))
  mat_spec = pl.BlockSpec((D, D), lambda i: (0, 0))
  if residual is None:
    return pl.pallas_call(
        _tc_layer_kernel,
        out_shape=jax.ShapeDtypeStruct((N, D), jnp.float32),
        grid=(N // BR,),
        in_specs=[row_spec, row_spec, mat_spec, vec_spec, vec_spec, vec_spec],
        out_specs=row_spec,
    )(p0, p1, w, b.reshape(1, D), gamma.reshape(1, D), beta.reshape(1, D))
  return pl.pallas_call(
      _tc_layer_final_kernel,
      out_shape=jax.ShapeDtypeStruct((N, D), jnp.float32),
      grid=(N // BR,),
      in_specs=[row_spec, row_spec, mat_spec, vec_spec, vec_spec, vec_spec,
                row_spec, row_spec],
      out_specs=row_spec,
  )(p0, p1, w, b.reshape(1, D), gamma.reshape(1, D), beta.reshape(1, D),
    residual, emb)


@jax.jit
def kernel(node_features, edge_index, edge_weight, W_proj, b_proj,
           W1, b1, W2, b2, gamma, beta, item_emb):
  row = edge_index[0]
  col = edge_index[1]
  pad = E_PAD - E
  zi = jnp.zeros((pad,), jnp.int32)
  row2d = jnp.concatenate([row, zi]).reshape(NW * CPW, CHUNK)
  col2d = jnp.concatenate([col, zi]).reshape(NW * CPW, CHUNK)
  ew_pad = jnp.concatenate([edge_weight, jnp.zeros((pad,), jnp.float32)])
  ew_exp = jnp.broadcast_to(ew_pad[:, None], (E_PAD, 16)).reshape(
      NW * CPW, 8, 128)

  h = _tc_proj(node_features, W_proj, b_proj)
  residual = h

  parts = _sc_aggregate(h, row2d, col2d, ew_exp)
  h = _tc_layer(parts, W1, b1, gamma, beta)

  parts = _sc_aggregate(h, row2d, col2d, ew_exp)
  out = _tc_layer(parts, W2, b2, gamma, beta, residual=residual, emb=item_emb)
  return out
